# baseline (speedup 1.0000x reference)
import functools

import jax
import jax.numpy as jnp
from jax import lax
from jax.experimental import pallas as pl
from jax.experimental.pallas import tpu as pltpu
from jax.experimental.pallas import tpu_sc as plsc

_NBUF = 2
_CHUNK = 208


@functools.partial(jax.jit, static_argnums=(2, 3))
def _embedding_lookup(idx_flat, wq, B, F):
    n = idx_flat.shape[0]
    VQ, DQ = wq.shape
    D = 32
    RPQ = DQ // D
    info = plsc.get_sparse_core_info()
    NC, NS = info.num_cores, info.num_subcores
    NW = NC * NS
    assert n % NW == 0
    b_per_w = n // NW
    C = _CHUNK
    NBUF = _NBUF
    assert b_per_w % (C * NBUF) == 0 and C % F == 0 and C % 16 == 0
    n_chunks = b_per_w // C
    bpc = C // F

    mesh = plsc.VectorSubcoreMesh(core_axis_name="c", subcore_axis_name="s")

    @functools.partial(
        pl.kernel,
        mesh=mesh,
        out_type=jax.ShapeDtypeStruct((B, F, D), jnp.float32),
        scratch_types=[
            pltpu.VMEM((b_per_w,), jnp.int32),
            *[pltpu.VMEM((C,), jnp.int32) for _ in range(NBUF)],
            *[pltpu.VMEM((C, DQ), jnp.float32) for _ in range(NBUF)],
            *[pltpu.VMEM((C, D), jnp.float32) for _ in range(NBUF)],
            *[pltpu.SemaphoreType.DMA for _ in range(2 * NBUF)],
        ],
    )
    def emb(table_hbm, idx_hbm, out_hbm, idx_v, *bufs_and_sems):
        idx4 = bufs_and_sems[:NBUF]
        rows = bufs_and_sems[NBUF : 2 * NBUF]
        ext = bufs_and_sems[2 * NBUF : 3 * NBUF]
        gsem = bufs_and_sems[3 * NBUF : 4 * NBUF]
        ssem = bufs_and_sems[4 * NBUF :]
        wid = lax.axis_index("s") * NC + lax.axis_index("c")
        base = wid * b_per_w

        def prep(j, b):
            for v in range(C // 16):
                idx4[b][pl.ds(v * 16, 16)] = lax.shift_right_logical(
                    idx_v[pl.ds(j * C + v * 16, 16)], RPQ - 2
                )

        def gather(b):
            return pltpu.make_async_copy(
                table_hbm.at[idx4[b]], rows[b], gsem[b]
            )

        def store(j, b, k):
            bb = (base + j * C) // F + k
            return pltpu.make_async_copy(
                ext[b].at[pl.ds(k * F, F), :], out_hbm.at[bb], ssem[b]
            )

        pltpu.sync_copy(idx_hbm.at[pl.ds(base, b_per_w)], idx_v)

        for b in range(NBUF):
            prep(b, b)
            gather(b).start()

        def step(g, carry):
            for b in range(NBUF):
                j = g * NBUF + b
                gather(b).wait()

                @pl.when(j >= NBUF)
                def _():
                    for k in range(bpc):
                        store(j - NBUF, b, k).wait()

                for v in range(C // 16):
                    offv = (idx_v[pl.ds(j * C + v * 16, 16)] & (RPQ - 1)) * D
                    for u in range(16):
                        off = offv[u]
                        r = v * 16 + u
                        ext[b][r, pl.ds(0, 16)] = rows[b][r, pl.ds(off, 16)]
                        ext[b][r, pl.ds(16, 16)] = rows[b][
                            r, pl.ds(off + 16, 16)
                        ]

                for k in range(bpc):
                    store(j, b, k).start()
                jn = j + NBUF

                @pl.when(jn < n_chunks)
                def _():
                    prep(jn, b)
                    gather(b).start()

            return carry

        lax.fori_loop(0, n_chunks // NBUF, step, 0)

        for b in range(NBUF):
            for k in range(bpc):
                store(n_chunks - NBUF + b, b, k).wait()

    return emb(wq, idx_flat)


def kernel(x, weight):
    B, F = x.shape
    V, D = weight.shape
    wq = weight.reshape(V // 4, D * 4)
    return _embedding_lookup(x.reshape(B * F).astype(jnp.int32), wq, B, F)


# speedup vs baseline: 1.1005x; 1.1005x over previous
import functools

import jax
import jax.numpy as jnp
from jax import lax
from jax.experimental import pallas as pl
from jax.experimental.pallas import tpu as pltpu
from jax.experimental.pallas import tpu_sc as plsc

_NBUF = 4
_CHUNK = 832


@jax.jit
def _embedding_lookup(idx_flat, weight):
    n = idx_flat.shape[0]
    V, D = weight.shape
    info = plsc.get_sparse_core_info()
    NC, NS = info.num_cores, info.num_subcores
    NW = NC * NS
    assert n % NW == 0
    b_per_w = n // NW
    C = _CHUNK
    NBUF = _NBUF
    assert b_per_w % (C * NBUF) == 0
    n_chunks = b_per_w // C

    mesh = plsc.VectorSubcoreMesh(core_axis_name="c", subcore_axis_name="s")

    @functools.partial(
        pl.kernel,
        mesh=mesh,
        out_type=jax.ShapeDtypeStruct((n, D), jnp.float32),
        scratch_types=[
            pltpu.VMEM((b_per_w,), jnp.int32),
            *[pltpu.VMEM((C, D), jnp.float32) for _ in range(NBUF)],
            *[pltpu.SemaphoreType.DMA for _ in range(2 * NBUF)],
        ],
        compiler_params=pltpu.CompilerParams(use_tc_tiling_on_sc=False),
    )
    def emb(table_hbm, idx_hbm, out_hbm, idx_v, *bufs_and_sems):
        rows = bufs_and_sems[:NBUF]
        gsem = bufs_and_sems[NBUF : 2 * NBUF]
        ssem = bufs_and_sems[2 * NBUF :]
        wid = lax.axis_index("s") * NC + lax.axis_index("c")
        base = wid * b_per_w

        def gather(j, b):
            return pltpu.make_async_copy(
                table_hbm.at[idx_v.at[pl.ds(j * C, C)]], rows[b], gsem[b]
            )

        def store(j, b):
            return pltpu.make_async_copy(
                rows[b], out_hbm.at[pl.ds(base + j * C, C)], ssem[b]
            )

        pltpu.sync_copy(idx_hbm.at[pl.ds(base, b_per_w)], idx_v)

        for b in range(NBUF):
            gather(b, b).start()

        def step(g, carry):
            for b in range(NBUF):
                j = g * NBUF + b
                gather(j, b).wait()
                store(j, b).start()
                jn = j + NBUF

                @pl.when(jn < n_chunks)
                def _():
                    store(j, b).wait()
                    gather(jn, b).start()

            return carry

        lax.fori_loop(0, n_chunks // NBUF, step, 0)

        for b in range(NBUF):
            store(n_chunks - NBUF + b, b).wait()

    return emb(weight, idx_flat)


def kernel(x, weight):
    B, F = x.shape
    out = _embedding_lookup(x.reshape(B * F).astype(jnp.int32), weight)
    return out.reshape(B, F, weight.shape[1])
